# baseline (speedup 1.0000x reference)
import functools

import jax
import jax.numpy as jnp
from jax import lax
from jax.experimental import pallas as pl
from jax.experimental.pallas import tpu as pltpu
from jax.experimental.pallas import tpu_sc as plsc

VOCAB = 1000000
DIM = 64
BATCH = 16384

NC = 2
NS = 16
L = 16
NW = NC * NS
BPW = BATCH // NW
CHUNK = 128
NCHUNK = BPW // CHUNK

_mesh = plsc.VectorSubcoreMesh(
    core_axis_name="c", subcore_axis_name="s", num_cores=NC, num_subcores=NS
)


@functools.partial(
    pl.kernel,
    out_type=jax.ShapeDtypeStruct((BATCH,), jnp.float32),
    mesh=_mesh,
    compiler_params=pltpu.CompilerParams(
        needs_layout_passes=False, use_tc_tiling_on_sc=False
    ),
    scratch_types=[
        pltpu.VMEM((NCHUNK, CHUNK), jnp.int32),
        pltpu.VMEM((NCHUNK, CHUNK), jnp.int32),
        pltpu.VMEM((BPW, DIM), jnp.float32),
        pltpu.VMEM((BPW, DIM), jnp.float32),
        pltpu.VMEM((BPW,), jnp.float32),
        pltpu.VMEM((BPW,), jnp.float32),
        pltpu.VMEM((BPW,), jnp.float32),
        pltpu.SemaphoreType.DMA,
    ],
)
def _glove_sc(i_hbm, j_hbm, wi_hbm, wj_hbm, bi_hbm, bj_hbm, out_hbm,
              iv, jv, wiv, wjv, biv, bjv, ov, sem):
    wid = lax.axis_index("s") * NC + lax.axis_index("c")
    base = wid * BPW

    for k in range(NCHUNK):
        pltpu.sync_copy(i_hbm.at[pl.ds(base + k * CHUNK, CHUNK)], iv.at[k])
        pltpu.sync_copy(j_hbm.at[pl.ds(base + k * CHUNK, CHUNK)], jv.at[k])

    copies = []
    for k in range(NCHUNK):
        dst = pl.ds(k * CHUNK, CHUNK)
        copies.append(pltpu.async_copy(wi_hbm.at[iv.at[k]], wiv.at[dst], sem))
        copies.append(pltpu.async_copy(wj_hbm.at[jv.at[k]], wjv.at[dst], sem))
        copies.append(pltpu.async_copy(bi_hbm.at[iv.at[k]], biv.at[dst], sem))
        copies.append(pltpu.async_copy(bj_hbm.at[jv.at[k]], bjv.at[dst], sem))
    for cp in copies:
        cp.wait()

    lane = lax.iota(jnp.int32, L)

    def group_body(g, _):
        r0 = g * L
        ridx = r0 + lane
        acc = biv[pl.ds(r0, L)] + bjv[pl.ds(r0, L)]
        for d in range(DIM):
            didx = jnp.full((L,), d, dtype=jnp.int32)
            a = plsc.load_gather(wiv, [ridx, didx])
            b = plsc.load_gather(wjv, [ridx, didx])
            acc = acc + a * b
        ov[pl.ds(r0, L)] = acc
        return 0

    lax.fori_loop(0, BPW // L, group_body, 0)

    pltpu.sync_copy(ov, out_hbm.at[pl.ds(base, BPW)])


def kernel(i_indices, j_indices, wi, wj, bi, bj):
    return _glove_sc(i_indices, j_indices, wi, wj,
                     bi.reshape(VOCAB), bj.reshape(VOCAB))
